# baseline (speedup 1.0000x reference)
import functools
import math

import jax
import jax.numpy as jnp
from jax import lax
from jax.experimental import pallas as pl
from jax.experimental.pallas import tpu as pltpu
from jax.experimental.pallas import tpu_sc as plsc

D_MODEL = 64
SCALE = math.sqrt(D_MODEL)

NC = 2
NS = 16
NW = NC * NS
L = 16
SUB = 128
CHUNK = 128
NSLOT = 5


def _make_sc_kernel(B4, J, D):
    n_chunks = (B4 * J) // CHUNK
    per_w = n_chunks // NW
    assert per_w * NW == n_chunks and per_w % NSLOT == 0
    groups = per_w // NSLOT
    bblk = B4 // SUB
    jblk = J // 8
    cpj = B4 // CHUNK
    mesh = plsc.VectorSubcoreMesh(core_axis_name="c", subcore_axis_name="s")

    @functools.partial(
        pl.kernel,
        mesh=mesh,
        out_type=jax.ShapeDtypeStruct((J, 8, bblk, 8, SUB), jnp.float32),
        scratch_types=[
            pltpu.VMEM((NSLOT, CHUNK // SUB, SUB), jnp.int32),
            pltpu.VMEM((NSLOT, CHUNK // SUB, SUB, D), jnp.float32),
            pltpu.VMEM((NSLOT, CHUNK // SUB, 8, 8, 131), jnp.float32),
            pltpu.SemaphoreType.DMA((NSLOT,)),
            pltpu.SemaphoreType.DMA((NSLOT,)),
            pltpu.SemaphoreType.DMA((NSLOT,)),
        ],
        compiler_params=pltpu.CompilerParams(
            use_tc_tiling_on_sc=False, needs_layout_passes=False
        ),
    )
    def k(table_hbm, idx_hbm, out_hbm, idxb, ibuf, obuf, xsem, isem, osem):
        wid = lax.axis_index("s") * NC + lax.axis_index("c")
        c0 = wid * per_w
        iota = lax.iota(jnp.int32, L)

        def chunk_coords(c):
            j = c // cpj
            cb0 = (c % cpj) * (CHUNK // SUB)
            return j, j // 8, j % 8, cb0

        def fire_idx(c, s):
            _, jb, ji, cb0 = chunk_coords(c)
            for t in range(CHUNK // SUB):
                pltpu.async_copy(
                    idx_hbm.at[jb, cb0 + t, ji], idxb.at[s, t], xsem.at[s]
                )

        def wait_idx(c, s):
            _, jb, ji, cb0 = chunk_coords(c)
            for t in range(CHUNK // SUB):
                pltpu.make_async_copy(
                    idx_hbm.at[jb, cb0 + t, ji], idxb.at[s, t], xsem.at[s]
                ).wait()

        def fire_in(s):
            for t in range(CHUNK // SUB):
                pltpu.async_copy(
                    table_hbm.at[idxb.at[s, t]], ibuf.at[s, t], isem.at[s]
                )

        def wait_in(s):
            for t in range(CHUNK // SUB):
                pltpu.make_async_copy(
                    table_hbm.at[idxb.at[s, t]], ibuf.at[s, t], isem.at[s]
                ).wait()

        def fire_out(c, s):
            j, _, _, cb0 = chunk_coords(c)
            for t in range(CHUNK // SUB):
                pltpu.async_copy(
                    obuf.at[s, t, :, :, pl.ds(0, SUB)],
                    out_hbm.at[j, :, cb0 + t],
                    osem.at[s],
                )

        def wait_out(c, s):
            j, _, _, cb0 = chunk_coords(c)
            for t in range(CHUNK // SUB):
                pltpu.make_async_copy(
                    obuf.at[s, t, :, :, pl.ds(0, SUB)],
                    out_hbm.at[j, :, cb0 + t],
                    osem.at[s],
                ).wait()

        rbv = [(iota + c0) >> 3 for c0 in range(0, D, L)]
        riv = [(iota + c0) & 7 for c0 in range(0, D, L)]

        def transpose_scale(s):
            for t in range(CHUNK // SUB):
                src2 = ibuf.at[s, t]
                dst3 = obuf.at[s, t]

                @pl.loop(0, SUB, step=4, unroll=4)
                def row_loop(r):
                    rsp = jnp.full((L,), r, jnp.int32)
                    for rr in range(4):
                        for q in range(D // L):
                            v = src2[r + rr, pl.ds(q * L, L)] * SCALE
                            plsc.store_scatter(
                                dst3, [rbv[q], riv[q], rsp + rr], v
                            )

        for s in range(NSLOT):
            fire_idx(c0 + s, s)
        for s in range(NSLOT):
            wait_idx(c0 + s, s)
            fire_in(s)

        for s in range(NSLOT):
            c = c0 + s
            wait_in(s)
            fire_idx(c + NSLOT, s)
            transpose_scale(s)
            fire_out(c, s)
            wait_idx(c + NSLOT, s)
            fire_in(s)

        @pl.loop(1, groups - 1)
        def group_loop(g):
            for s in range(NSLOT):
                c = c0 + g * NSLOT + s
                wait_in(s)
                fire_idx(c + NSLOT, s)
                wait_out(c - NSLOT, s)
                transpose_scale(s)
                fire_out(c, s)
                wait_idx(c + NSLOT, s)
                fire_in(s)

        for s in range(NSLOT):
            c = c0 + (groups - 1) * NSLOT + s
            wait_in(s)
            wait_out(c - NSLOT, s)
            transpose_scale(s)
            fire_out(c, s)
        for s in range(NSLOT):
            c = c0 + (groups - 1) * NSLOT + s
            wait_out(c, s)

    return k


def kernel(x, table):
    B4, J = x.shape
    D = table.shape[1]
    idx4 = (
        x.T.astype(jnp.int32)
        .reshape(J // 8, 8, B4 // SUB, SUB)
        .transpose(0, 2, 1, 3)
    )
    out5 = _make_sc_kernel(B4, J, D)(table, idx4)
    return out5.transpose(2, 4, 0, 1, 3).reshape(B4, J, D)


# speedup vs baseline: 1.0100x; 1.0100x over previous
import functools
import math

import jax
import jax.numpy as jnp
from jax import lax
from jax.experimental import pallas as pl
from jax.experimental.pallas import tpu as pltpu
from jax.experimental.pallas import tpu_sc as plsc

D_MODEL = 64
SCALE = math.sqrt(D_MODEL)

NC = 2
NS = 16
NW = NC * NS
L = 16
SUB = 128
CHUNK = 128
NSLOT = 4


def _make_sc_kernel(B4, J, D):
    n_chunks = (B4 * J) // CHUNK
    per_w = n_chunks // NW
    assert per_w * NW == n_chunks and per_w % NSLOT == 0
    groups = per_w // NSLOT
    bblk = B4 // SUB
    jblk = J // 8
    cpj = B4 // CHUNK
    mesh = plsc.VectorSubcoreMesh(core_axis_name="c", subcore_axis_name="s")

    @functools.partial(
        pl.kernel,
        mesh=mesh,
        out_type=jax.ShapeDtypeStruct((J, 8, bblk, 8, SUB), jnp.float32),
        scratch_types=[
            pltpu.VMEM((NSLOT, CHUNK // SUB, SUB), jnp.int32),
            pltpu.VMEM((NSLOT, CHUNK // SUB, SUB, D), jnp.float32),
            pltpu.VMEM((NSLOT, CHUNK // SUB, 8, 8, 131), jnp.float32),
            pltpu.SemaphoreType.DMA((NSLOT,)),
            pltpu.SemaphoreType.DMA((NSLOT,)),
            pltpu.SemaphoreType.DMA((NSLOT,)),
        ],
        compiler_params=pltpu.CompilerParams(
            use_tc_tiling_on_sc=False, needs_layout_passes=False
        ),
    )
    def k(table_hbm, idx_hbm, out_hbm, idxb, ibuf, obuf, xsem, isem, osem):
        wid = lax.axis_index("s") * NC + lax.axis_index("c")
        c0 = wid * per_w
        iota = lax.iota(jnp.int32, L)

        def chunk_coords(c):
            j = c // cpj
            cb0 = (c % cpj) * (CHUNK // SUB)
            return j, j // 8, j % 8, cb0

        def fire_idx(c, s):
            _, jb, ji, cb0 = chunk_coords(c)
            for t in range(CHUNK // SUB):
                pltpu.async_copy(
                    idx_hbm.at[jb, cb0 + t, ji], idxb.at[s, t], xsem.at[s]
                )

        def wait_idx(c, s):
            _, jb, ji, cb0 = chunk_coords(c)
            for t in range(CHUNK // SUB):
                pltpu.make_async_copy(
                    idx_hbm.at[jb, cb0 + t, ji], idxb.at[s, t], xsem.at[s]
                ).wait()

        def fire_in(s):
            for t in range(CHUNK // SUB):
                pltpu.async_copy(
                    table_hbm.at[idxb.at[s, t]], ibuf.at[s, t], isem.at[s]
                )

        def wait_in(s):
            for t in range(CHUNK // SUB):
                pltpu.make_async_copy(
                    table_hbm.at[idxb.at[s, t]], ibuf.at[s, t], isem.at[s]
                ).wait()

        def fire_out(c, s):
            j, _, _, cb0 = chunk_coords(c)
            for t in range(CHUNK // SUB):
                pltpu.async_copy(
                    obuf.at[s, t, :, :, pl.ds(0, SUB)],
                    out_hbm.at[j, :, cb0 + t],
                    osem.at[s],
                )

        def wait_out(c, s):
            j, _, _, cb0 = chunk_coords(c)
            for t in range(CHUNK // SUB):
                pltpu.make_async_copy(
                    obuf.at[s, t, :, :, pl.ds(0, SUB)],
                    out_hbm.at[j, :, cb0 + t],
                    osem.at[s],
                ).wait()

        rbv = [(iota + c0) >> 3 for c0 in range(0, D, L)]
        riv = [(iota + c0) & 7 for c0 in range(0, D, L)]

        def transpose_scale(s):
            for t in range(CHUNK // SUB):
                src2 = ibuf.at[s, t]
                dst3 = obuf.at[s, t]

                @pl.loop(0, SUB, step=4, unroll=4)
                def row_loop(r):
                    rsp = jnp.full((L,), r, jnp.int32)
                    for rr in range(4):
                        for q in range(D // L):
                            v = src2[r + rr, pl.ds(q * L, L)] * SCALE
                            plsc.store_scatter(
                                dst3, [rbv[q], riv[q], rsp + rr], v
                            )

        for s in range(NSLOT):
            fire_idx(c0 + s, s)
        for s in range(NSLOT):
            wait_idx(c0 + s, s)
            fire_in(s)

        for s in range(NSLOT):
            c = c0 + s
            wait_in(s)
            fire_idx(c + NSLOT, s)
            transpose_scale(s)
            fire_out(c, s)
            wait_idx(c + NSLOT, s)
            fire_in(s)

        @pl.loop(1, groups - 1)
        def group_loop(g):
            for s in range(NSLOT):
                c = c0 + g * NSLOT + s
                wait_in(s)
                fire_idx(c + NSLOT, s)
                wait_out(c - NSLOT, s)
                transpose_scale(s)
                fire_out(c, s)
                wait_idx(c + NSLOT, s)
                fire_in(s)

        for s in range(NSLOT):
            c = c0 + (groups - 1) * NSLOT + s
            wait_in(s)
            wait_out(c - NSLOT, s)
            transpose_scale(s)
            fire_out(c, s)
        for s in range(NSLOT):
            c = c0 + (groups - 1) * NSLOT + s
            wait_out(c, s)

    return k


def kernel(x, table):
    B4, J = x.shape
    D = table.shape[1]
    idx4 = (
        x.T.astype(jnp.int32)
        .reshape(J // 8, 8, B4 // SUB, SUB)
        .transpose(0, 2, 1, 3)
    )
    out5 = _make_sc_kernel(B4, J, D)(table, idx4)
    return out5.transpose(2, 4, 0, 1, 3).reshape(B4, J, D)
